# baseline (speedup 1.0000x reference)
import jax
import jax.numpy as jnp
from jax import lax
from jax.experimental import pallas as pl
from jax.experimental.pallas import tpu as pltpu
from jax.experimental.pallas import tpu_sc as plsc

_N_NODES = 10000
_N_HEDGES = 5000
_E = 320000
_D = 128

_NC = 2
_NS = 16
_NW = _NC * _NS
_EPW = _E // _NW
_CH = 80
_NCH = _EPW // _CH
_HE_PAD = 5120
_NO_PAD = 10240
_HE_PT = _HE_PAD // _NS
_NO_PT = _NO_PAD // _NS

_mesh = plsc.VectorSubcoreMesh(core_axis_name="c", subcore_axis_name="s")
_f32 = jnp.float32


_DK = 5
_DGRP = _DK * _CH
_DNIT = _EPW // _DGRP


def _make_sc_deg(rows_pad):
    rows_pt = rows_pad // _NS

    def body(idx, zrow, ones, dg,
             dacc, sa, sb, onev,
             semsx, x0, x1, x2, x3, x4):
        ssems = (x0, x1, x2, x3, x4)
        cid = lax.axis_index("c")
        sid = lax.axis_index("s")
        base = (sid * _NC + cid) * _EPW
        r0 = sid * rows_pt

        def fire_idx(off, sbuf):
            for j in range(_DK):
                pltpu.async_copy(idx.at[pl.ds(off + j * _CH, _CH)],
                                 sbuf.at[j], semsx)

        def run_iter(sbuf):
            for j in range(_DK):
                pltpu.make_async_copy(idx.at[pl.ds(0, _CH)],
                                      sbuf.at[j], semsx).wait()
            descs = [
                pltpu.async_copy(onev, dacc.at[sbuf.at[j]], ssems[j], add=True)
                for j in range(_DK)
            ]
            for d in descs:
                d.wait()

        fire_idx(base, sa)
        pltpu.sync_copy(zrow.at[pl.ds(0, rows_pt)], dacc.at[pl.ds(r0, rows_pt)])
        pltpu.sync_copy(ones, onev)
        plsc.subcore_barrier()

        def step(k, carry):
            offa = base + (2 * k) * _DGRP
            fire_idx(offa + _DGRP, sb)
            run_iter(sa)
            fire_idx(offa + 2 * _DGRP, sa)
            run_iter(sb)
            return carry

        lax.fori_loop(0, (_DNIT - 1) // 2, step, 0)
        run_iter(sa)
        plsc.subcore_barrier()
        pltpu.sync_copy(dacc.at[pl.ds(r0, rows_pt)], dg.at[cid, pl.ds(r0, rows_pt)])

    return pl.kernel(
        body,
        out_type=jax.ShapeDtypeStruct((_NC, rows_pad, _D), _f32),
        mesh=_mesh,
        scratch_types=(
            [
                pltpu.VMEM_SHARED((rows_pad, _D), _f32),
                pltpu.VMEM((_DK, _CH), jnp.int32),
                pltpu.VMEM((_DK, _CH), jnp.int32),
                pltpu.VMEM((_CH, _D), _f32),
            ]
            + [pltpu.SemaphoreType.DMA for _ in range(_DK + 1)]
        ),
    )


_sc_deg_e = _make_sc_deg(_HE_PAD)
_sc_deg_n = _make_sc_deg(_NO_PAD)


def _make_sc_phase(gather_first, acc_rows, nk, ch):
    acc_pt = acc_rows // _NS
    grp = nk * ch
    nit = _EPW // grp
    tail = _EPW - nit * grp
    assert ch % 8 == 0 and ch % 128 != 0 and tail % 80 == 0 and nit % 2 == 1

    def body(table, nidx, hidx, zrow,
             part,
             acc, ga, gb,
             *rest):
        rows = rest[:nk]
        sa = rest[nk:2 * nk]
        sb = rest[2 * nk:3 * nk]
        st = rest[3 * nk]
        semg, semsx = rest[3 * nk + 1], rest[3 * nk + 2]
        sems = rest[3 * nk + 3:3 * nk + 3 + nk]
        ssems = rest[3 * nk + 3 + nk:]
        gidx, sidx = (nidx, hidx) if gather_first else (hidx, nidx)
        cid = lax.axis_index("c")
        sid = lax.axis_index("s")
        base = (sid * _NC + cid) * _EPW
        a0 = sid * acc_pt

        def fire_idx(off, gbuf, sbuf):
            pltpu.async_copy(gidx.at[pl.ds(off, grp)], gbuf, semg)
            for j in range(nk):
                pltpu.async_copy(sidx.at[pl.ds(off + j * ch, ch)],
                                 sbuf[j], semsx)

        def drain_idx(gbuf, sbuf):
            pltpu.make_async_copy(gidx.at[pl.ds(0, grp)], gbuf, semg).wait()
            for j in range(nk):
                pltpu.make_async_copy(sidx.at[pl.ds(0, ch)],
                                      sbuf[j], semsx).wait()

        def run_iter(gbuf, sbuf):
            drain_idx(gbuf, sbuf)
            return [
                pltpu.async_copy(table.at[gbuf.at[pl.ds(j * ch, ch)]],
                                 rows[j], sems[j])
                for j in range(nk)
            ]

        def consume(descs, sbuf):
            sdescs = []
            for j in range(nk):
                descs[j].wait()
                sdescs.append(
                    pltpu.async_copy(rows[j], acc.at[sbuf[j]], ssems[j],
                                     add=True))
            for d in sdescs:
                d.wait()

        fire_idx(base, ga, sa)
        pltpu.sync_copy(zrow.at[pl.ds(0, acc_pt)], acc.at[pl.ds(a0, acc_pt)])
        plsc.subcore_barrier()

        def step(k, carry):
            offa = base + (2 * k) * grp
            descs = run_iter(ga, sa)
            fire_idx(offa + grp, gb, sb)
            consume(descs, sa)
            descs = run_iter(gb, sb)
            fire_idx(offa + 2 * grp, ga, sa)
            consume(descs, sb)
            return carry

        lax.fori_loop(0, (nit - 1) // 2, step, 0)
        descs = run_iter(ga, sa)
        consume(descs, sa)
        for t in range(tail // 80):
            toff = base + nit * grp + t * 80
            pltpu.sync_copy(gidx.at[pl.ds(toff, 80)], gb.at[pl.ds(0, 80)])
            pltpu.sync_copy(sidx.at[pl.ds(toff, 80)], st)
            pltpu.async_copy(table.at[gb.at[pl.ds(0, 80)]],
                             rows[0].at[pl.ds(0, 80)], sems[0]).wait()
            pltpu.sync_copy(rows[0].at[pl.ds(0, 80)], acc.at[st],
                            add=True)

        plsc.subcore_barrier()
        pltpu.sync_copy(acc.at[pl.ds(a0, acc_pt)], part.at[cid, pl.ds(a0, acc_pt)])

    return pl.kernel(
        body,
        out_type=jax.ShapeDtypeStruct((_NC, acc_rows, _D), _f32),
        mesh=_mesh,
        scratch_types=(
            [
                pltpu.VMEM_SHARED((acc_rows, _D), _f32),
                pltpu.VMEM((grp,), jnp.int32),
                pltpu.VMEM((grp,), jnp.int32),
            ]
            + [pltpu.VMEM((ch, _D), _f32) for _ in range(nk)]
            + [pltpu.VMEM((ch,), jnp.int32) for _ in range(2 * nk)]
            + [pltpu.VMEM((80,), jnp.int32)]
            + [pltpu.SemaphoreType.DMA for _ in range(2 * nk + 2)]
        ),
    )


_sc_phase_a = _make_sc_phase(True, _HE_PAD, 2, 320)
_sc_phase_b = _make_sc_phase(False, _NO_PAD, 2, 160)


_RB = 2000
_EB = 640
_NB = 640


def _fuse_body(xo_ref, xd_ref, wf_ref, bf_ref, w1_ref, h1_ref):
    xo = xo_ref[...]
    xd = xd_ref[...]
    wf = wf_ref[...]
    bf = bf_ref[...]
    so = jax.nn.sigmoid(jnp.dot(xo, wf, preferred_element_type=_f32) + bf)
    sd = jax.nn.sigmoid(jnp.dot(xd, wf, preferred_element_type=_f32) + bf)
    x = so * xo + sd * xd
    h1_ref[...] = jnp.dot(x, w1_ref[...], preferred_element_type=_f32)


def _tc_fuse(xo, xd, wf, bf, w1):
    return pl.pallas_call(
        _fuse_body,
        grid=(_N_NODES // _RB,),
        in_specs=[
            pl.BlockSpec((_RB, _D), lambda i: (i, 0)),
            pl.BlockSpec((_RB, _D), lambda i: (i, 0)),
            pl.BlockSpec((_D, _D), lambda i: (0, 0)),
            pl.BlockSpec((1, _D), lambda i: (0, 0)),
            pl.BlockSpec((_D, _D), lambda i: (0, 0)),
        ],
        out_specs=pl.BlockSpec((_RB, _D), lambda i: (i, 0)),
        out_shape=jax.ShapeDtypeStruct((_N_NODES, _D), _f32),
    )(xo, xd, wf, bf, w1)


def _edges_body(ep_ref, de_ref, en_ref):
    s = ep_ref[0, :, :] + ep_ref[1, :, :]
    deg = jnp.maximum(de_ref[0, :, 0:1] + de_ref[1, :, 0:1], 1.0)
    en_ref[...] = s / deg


def _tc_edges(ep, de):
    return pl.pallas_call(
        _edges_body,
        grid=(_HE_PAD // _EB,),
        in_specs=[
            pl.BlockSpec((_NC, _EB, _D), lambda i: (0, i, 0)),
            pl.BlockSpec((_NC, _EB, _D), lambda i: (0, i, 0)),
        ],
        out_specs=pl.BlockSpec((_EB, _D), lambda i: (i, 0)),
        out_shape=jax.ShapeDtypeStruct((_HE_PAD, _D), _f32),
    )(ep, de)


def _nodes_mm_body(np_ref, dn_ref, w_ref, h_ref):
    s = np_ref[0, :, :] + np_ref[1, :, :]
    deg = jnp.maximum(dn_ref[0, :, 0:1] + dn_ref[1, :, 0:1], 1.0)
    x = jnp.maximum(s / deg, 0.0)
    h_ref[...] = jnp.dot(x, w_ref[...], preferred_element_type=_f32)


def _tc_nodes_mm(npart, dn, w):
    return pl.pallas_call(
        _nodes_mm_body,
        grid=(_NO_PAD // _NB,),
        in_specs=[
            pl.BlockSpec((_NC, _NB, _D), lambda i: (0, i, 0)),
            pl.BlockSpec((_NC, _NB, _D), lambda i: (0, i, 0)),
            pl.BlockSpec((_D, _D), lambda i: (0, 0)),
        ],
        out_specs=pl.BlockSpec((_NB, _D), lambda i: (i, 0)),
        out_shape=jax.ShapeDtypeStruct((_NO_PAD, _D), _f32),
    )(npart, dn, w)


def _nodes_relu_body(np_ref, dn_ref, x_ref):
    s = np_ref[0, :, :] + np_ref[1, :, :]
    deg = jnp.maximum(dn_ref[0, :, 0:1] + dn_ref[1, :, 0:1], 1.0)
    x_ref[...] = jnp.maximum(s / deg, 0.0)


def _tc_nodes_relu(npart, dn):
    return pl.pallas_call(
        _nodes_relu_body,
        grid=(_NO_PAD // _NB,),
        in_specs=[
            pl.BlockSpec((_NC, _NB, _D), lambda i: (0, i, 0)),
            pl.BlockSpec((_NC, _NB, _D), lambda i: (0, i, 0)),
        ],
        out_specs=pl.BlockSpec((_NB, _D), lambda i: (i, 0)),
        out_shape=jax.ShapeDtypeStruct((_NO_PAD, _D), _f32),
    )(npart, dn)


def kernel(x_ori, x_dy, node_idx, hedge_idx, W_fuse, b_fuse, W1, W2):
    nidx = node_idx.astype(jnp.int32)
    hidx = hedge_idx.astype(jnp.int32)
    zrow = jnp.zeros((_NO_PT, _D), _f32)
    ones = jnp.ones((_CH, _D), _f32)

    h1 = _tc_fuse(x_ori, x_dy, W_fuse, b_fuse.reshape(1, _D), W1)
    de = _sc_deg_e(hidx, zrow, ones)
    dn = _sc_deg_n(nidx, zrow, ones)
    ep1 = _sc_phase_a(h1, nidx, hidx, zrow)
    en1 = _tc_edges(ep1, de)
    np1 = _sc_phase_b(en1, nidx, hidx, zrow)
    h2 = _tc_nodes_mm(np1, dn, W2)
    ep2 = _sc_phase_a(h2, nidx, hidx, zrow)
    en2 = _tc_edges(ep2, de)
    np2 = _sc_phase_b(en2, nidx, hidx, zrow)
    x_out = _tc_nodes_relu(np2, dn)
    return x_out[:_N_NODES], en2[:_N_HEDGES]


# speedup vs baseline: 1.0328x; 1.0328x over previous
import jax
import jax.numpy as jnp
from jax import lax
from jax.experimental import pallas as pl
from jax.experimental.pallas import tpu as pltpu
from jax.experimental.pallas import tpu_sc as plsc

_N_NODES = 10000
_N_HEDGES = 5000
_E = 320000
_D = 128

_NC = 2
_NS = 16
_NW = _NC * _NS
_EPW = _E // _NW
_CH = 80
_NCH = _EPW // _CH
_HE_PAD = 5120
_NO_PAD = 10240
_HE_PT = _HE_PAD // _NS
_NO_PT = _NO_PAD // _NS

_mesh = plsc.VectorSubcoreMesh(core_axis_name="c", subcore_axis_name="s")
_f32 = jnp.float32


_DK = 5
_DGRP = _DK * _CH
_DNIT = _EPW // _DGRP


def _make_sc_deg(rows_pad):
    rows_pt = rows_pad // _NS

    def body(idx, zrow, ones, dg,
             dacc, sa, sb, onev,
             semsx, x0, x1, x2, x3, x4):
        ssems = (x0, x1, x2, x3, x4)
        cid = lax.axis_index("c")
        sid = lax.axis_index("s")
        base = (sid * _NC + cid) * _EPW
        r0 = sid * rows_pt

        def fire_idx(off, sbuf):
            for j in range(_DK):
                pltpu.async_copy(idx.at[pl.ds(off + j * _CH, _CH)],
                                 sbuf.at[j], semsx)

        def run_iter(sbuf):
            for j in range(_DK):
                pltpu.make_async_copy(idx.at[pl.ds(0, _CH)],
                                      sbuf.at[j], semsx).wait()
            descs = [
                pltpu.async_copy(onev, dacc.at[sbuf.at[j]], ssems[j], add=True)
                for j in range(_DK)
            ]
            for d in descs:
                d.wait()

        fire_idx(base, sa)
        pltpu.sync_copy(zrow.at[pl.ds(0, rows_pt)], dacc.at[pl.ds(r0, rows_pt)])
        pltpu.sync_copy(ones, onev)
        plsc.subcore_barrier()

        def step(k, carry):
            offa = base + (2 * k) * _DGRP
            fire_idx(offa + _DGRP, sb)
            run_iter(sa)
            fire_idx(offa + 2 * _DGRP, sa)
            run_iter(sb)
            return carry

        lax.fori_loop(0, (_DNIT - 1) // 2, step, 0)
        run_iter(sa)
        plsc.subcore_barrier()
        pltpu.sync_copy(dacc.at[pl.ds(r0, rows_pt)], dg.at[cid, pl.ds(r0, rows_pt)])

    return pl.kernel(
        body,
        out_type=jax.ShapeDtypeStruct((_NC, rows_pad, _D), _f32),
        mesh=_mesh,
        scratch_types=(
            [
                pltpu.VMEM_SHARED((rows_pad, _D), _f32),
                pltpu.VMEM((_DK, _CH), jnp.int32),
                pltpu.VMEM((_DK, _CH), jnp.int32),
                pltpu.VMEM((_CH, _D), _f32),
            ]
            + [pltpu.SemaphoreType.DMA for _ in range(_DK + 1)]
        ),
    )


_sc_deg_e = _make_sc_deg(_HE_PAD)
_sc_deg_n = _make_sc_deg(_NO_PAD)


def _make_sc_phase(gather_first, acc_rows, nk, ch):
    acc_pt = acc_rows // _NS
    grp = nk * ch
    nit = _EPW // grp
    tail = _EPW - nit * grp
    assert ch % 8 == 0 and ch % 128 != 0 and tail % 80 == 0 and nit % 2 == 1

    def body(table, nidx, hidx, zrow,
             part,
             acc, ga, gb,
             *rest):
        rows = rest[:nk]
        sa = rest[nk:2 * nk]
        sb = rest[2 * nk:3 * nk]
        st = rest[3 * nk]
        semg, semsx = rest[3 * nk + 1], rest[3 * nk + 2]
        sems = rest[3 * nk + 3:3 * nk + 3 + nk]
        ssems = rest[3 * nk + 3 + nk:]
        gidx, sidx = (nidx, hidx) if gather_first else (hidx, nidx)
        cid = lax.axis_index("c")
        sid = lax.axis_index("s")
        base = (sid * _NC + cid) * _EPW
        a0 = sid * acc_pt

        def fire_idx(off, gbuf, sbuf):
            pltpu.async_copy(gidx.at[pl.ds(off, grp)], gbuf, semg)
            for j in range(nk):
                pltpu.async_copy(sidx.at[pl.ds(off + j * ch, ch)],
                                 sbuf[j], semsx)

        def drain_idx(gbuf, sbuf):
            pltpu.make_async_copy(gidx.at[pl.ds(0, grp)], gbuf, semg).wait()
            for j in range(nk):
                pltpu.make_async_copy(sidx.at[pl.ds(0, ch)],
                                      sbuf[j], semsx).wait()

        def run_iter(gbuf, sbuf):
            drain_idx(gbuf, sbuf)
            return [
                pltpu.async_copy(table.at[gbuf.at[pl.ds(j * ch, ch)]],
                                 rows[j], sems[j])
                for j in range(nk)
            ]

        def consume(descs, sbuf):
            sdescs = []
            for j in range(nk):
                descs[j].wait()
                sdescs.append(
                    pltpu.async_copy(rows[j], acc.at[sbuf[j]], ssems[j],
                                     add=True))
            for d in sdescs:
                d.wait()

        fire_idx(base, ga, sa)
        pltpu.sync_copy(zrow.at[pl.ds(0, acc_pt)], acc.at[pl.ds(a0, acc_pt)])
        plsc.subcore_barrier()

        def step(k, carry):
            offa = base + (2 * k) * grp
            descs = run_iter(ga, sa)
            fire_idx(offa + grp, gb, sb)
            consume(descs, sa)
            descs = run_iter(gb, sb)
            fire_idx(offa + 2 * grp, ga, sa)
            consume(descs, sb)
            return carry

        lax.fori_loop(0, (nit - 1) // 2, step, 0)
        descs = run_iter(ga, sa)
        consume(descs, sa)
        for t in range(tail // 80):
            toff = base + nit * grp + t * 80
            pltpu.sync_copy(gidx.at[pl.ds(toff, 80)], gb.at[pl.ds(0, 80)])
            pltpu.sync_copy(sidx.at[pl.ds(toff, 80)], st)
            pltpu.async_copy(table.at[gb.at[pl.ds(0, 80)]],
                             rows[0].at[pl.ds(0, 80)], sems[0]).wait()
            pltpu.sync_copy(rows[0].at[pl.ds(0, 80)], acc.at[st],
                            add=True)

        plsc.subcore_barrier()
        pltpu.sync_copy(acc.at[pl.ds(a0, acc_pt)], part.at[cid, pl.ds(a0, acc_pt)])

    return pl.kernel(
        body,
        out_type=jax.ShapeDtypeStruct((_NC, acc_rows, _D), _f32),
        mesh=_mesh,
        scratch_types=(
            [
                pltpu.VMEM_SHARED((acc_rows, _D), _f32),
                pltpu.VMEM((grp,), jnp.int32),
                pltpu.VMEM((grp,), jnp.int32),
            ]
            + [pltpu.VMEM((ch, _D), _f32) for _ in range(nk)]
            + [pltpu.VMEM((ch,), jnp.int32) for _ in range(2 * nk)]
            + [pltpu.VMEM((80,), jnp.int32)]
            + [pltpu.SemaphoreType.DMA for _ in range(2 * nk + 2)]
        ),
    )


_sc_phase_a = _make_sc_phase(True, _HE_PAD, 5, 80)
_sc_phase_b = _make_sc_phase(False, _NO_PAD, 4, 80)


_RB = 2000
_EB = 640
_NB = 640


def _fuse_body(xo_ref, xd_ref, wf_ref, bf_ref, w1_ref, h1_ref):
    xo = xo_ref[...]
    xd = xd_ref[...]
    wf = wf_ref[...]
    bf = bf_ref[...]
    so = jax.nn.sigmoid(jnp.dot(xo, wf, preferred_element_type=_f32) + bf)
    sd = jax.nn.sigmoid(jnp.dot(xd, wf, preferred_element_type=_f32) + bf)
    x = so * xo + sd * xd
    h1_ref[...] = jnp.dot(x, w1_ref[...], preferred_element_type=_f32)


def _tc_fuse(xo, xd, wf, bf, w1):
    return pl.pallas_call(
        _fuse_body,
        grid=(_N_NODES // _RB,),
        in_specs=[
            pl.BlockSpec((_RB, _D), lambda i: (i, 0)),
            pl.BlockSpec((_RB, _D), lambda i: (i, 0)),
            pl.BlockSpec((_D, _D), lambda i: (0, 0)),
            pl.BlockSpec((1, _D), lambda i: (0, 0)),
            pl.BlockSpec((_D, _D), lambda i: (0, 0)),
        ],
        out_specs=pl.BlockSpec((_RB, _D), lambda i: (i, 0)),
        out_shape=jax.ShapeDtypeStruct((_N_NODES, _D), _f32),
    )(xo, xd, wf, bf, w1)


def _edges_body(ep_ref, de_ref, en_ref):
    s = ep_ref[0, :, :] + ep_ref[1, :, :]
    deg = jnp.maximum(de_ref[0, :, 0:1] + de_ref[1, :, 0:1], 1.0)
    en_ref[...] = s / deg


def _tc_edges(ep, de):
    return pl.pallas_call(
        _edges_body,
        grid=(_HE_PAD // _EB,),
        in_specs=[
            pl.BlockSpec((_NC, _EB, _D), lambda i: (0, i, 0)),
            pl.BlockSpec((_NC, _EB, _D), lambda i: (0, i, 0)),
        ],
        out_specs=pl.BlockSpec((_EB, _D), lambda i: (i, 0)),
        out_shape=jax.ShapeDtypeStruct((_HE_PAD, _D), _f32),
    )(ep, de)


def _nodes_mm_body(np_ref, dn_ref, w_ref, h_ref):
    s = np_ref[0, :, :] + np_ref[1, :, :]
    deg = jnp.maximum(dn_ref[0, :, 0:1] + dn_ref[1, :, 0:1], 1.0)
    x = jnp.maximum(s / deg, 0.0)
    h_ref[...] = jnp.dot(x, w_ref[...], preferred_element_type=_f32)


def _tc_nodes_mm(npart, dn, w):
    return pl.pallas_call(
        _nodes_mm_body,
        grid=(_NO_PAD // _NB,),
        in_specs=[
            pl.BlockSpec((_NC, _NB, _D), lambda i: (0, i, 0)),
            pl.BlockSpec((_NC, _NB, _D), lambda i: (0, i, 0)),
            pl.BlockSpec((_D, _D), lambda i: (0, 0)),
        ],
        out_specs=pl.BlockSpec((_NB, _D), lambda i: (i, 0)),
        out_shape=jax.ShapeDtypeStruct((_NO_PAD, _D), _f32),
    )(npart, dn, w)


def _nodes_relu_body(np_ref, dn_ref, x_ref):
    s = np_ref[0, :, :] + np_ref[1, :, :]
    deg = jnp.maximum(dn_ref[0, :, 0:1] + dn_ref[1, :, 0:1], 1.0)
    x_ref[...] = jnp.maximum(s / deg, 0.0)


def _tc_nodes_relu(npart, dn):
    return pl.pallas_call(
        _nodes_relu_body,
        grid=(_NO_PAD // _NB,),
        in_specs=[
            pl.BlockSpec((_NC, _NB, _D), lambda i: (0, i, 0)),
            pl.BlockSpec((_NC, _NB, _D), lambda i: (0, i, 0)),
        ],
        out_specs=pl.BlockSpec((_NB, _D), lambda i: (i, 0)),
        out_shape=jax.ShapeDtypeStruct((_NO_PAD, _D), _f32),
    )(npart, dn)


def kernel(x_ori, x_dy, node_idx, hedge_idx, W_fuse, b_fuse, W1, W2):
    nidx = node_idx.astype(jnp.int32)
    hidx = hedge_idx.astype(jnp.int32)
    zrow = jnp.zeros((_NO_PT, _D), _f32)
    ones = jnp.ones((_CH, _D), _f32)

    h1 = _tc_fuse(x_ori, x_dy, W_fuse, b_fuse.reshape(1, _D), W1)
    de = _sc_deg_e(hidx, zrow, ones)
    dn = _sc_deg_n(nidx, zrow, ones)
    ep1 = _sc_phase_a(h1, nidx, hidx, zrow)
    en1 = _tc_edges(ep1, de)
    np1 = _sc_phase_b(en1, nidx, hidx, zrow)
    h2 = _tc_nodes_mm(np1, dn, W2)
    ep2 = _sc_phase_a(h2, nidx, hidx, zrow)
    en2 = _tc_edges(ep2, de)
    np2 = _sc_phase_b(en2, nidx, hidx, zrow)
    x_out = _tc_nodes_relu(np2, dn)
    return x_out[:_N_NODES], en2[:_N_HEDGES]
